# baseline (speedup 1.0000x reference)
import functools

import jax
import jax.numpy as jnp
from jax import lax
from jax.experimental import pallas as pl
from jax.experimental.pallas import tpu as pltpu
from jax.experimental.pallas import tpu_sc as plsc

N = 10000
E = 320000
D = 128
EPS_BN = 1e-5

NC = 2
NS = 16
NW = NC * NS

CH = 128
NCH = 80
TOT_CH = NCH * NW
E_PAD = TOT_CH * CH
NCH0 = 80
NCH1 = 80
NCH_MAX = max(NCH0, NCH1)
N_PAD = 10240
ROWS_PER_TILE = N_PAD // NS

_mesh = plsc.VectorSubcoreMesh(core_axis_name="c", subcore_axis_name="s")


def _zero_rows(zbuf, lanes_per_row):
    zv = jnp.zeros((16,), jnp.float32)

    def body(i, _):
        for k in range(lanes_per_row // 16):
            zbuf[i, pl.ds(k * 16, 16)] = zv
        return 0

    lax.fori_loop(0, CH, body, 0)


@functools.partial(
    pl.kernel,
    out_type=jax.ShapeDtypeStruct((NC, N_PAD, 16), jnp.float32),
    mesh=_mesh,
    scratch_types=[
        pltpu.VMEM((NCH, CH), jnp.int32),
        pltpu.VMEM((CH, 16), jnp.float32),
        pltpu.VMEM((CH, 16), jnp.float32),
        pltpu.VMEM_SHARED((N_PAD, 16), jnp.float32),
        pltpu.SemaphoreType.DMA,
    ],
)
def _deg_kernel(dst_hbm, out_hbm, dst_v, ones_v, zbuf, acc, sem):
    c = lax.axis_index("c")
    s = lax.axis_index("s")
    w = c * NS + s

    ov = jnp.ones((16,), jnp.float32)

    def fill(i, _):
        ones_v[i, :] = ov
        return 0

    lax.fori_loop(0, CH, fill, 0)
    _zero_rows(zbuf, 16)
    for k in range(ROWS_PER_TILE // CH):
        pltpu.sync_copy(zbuf, acc.at[pl.ds(s * ROWS_PER_TILE + k * CH, CH)])
    plsc.subcore_barrier()

    pltpu.sync_copy(dst_hbm.at[w], dst_v)

    def edge_chunk(j, _):
        pltpu.sync_copy(ones_v, acc.at[dst_v.at[j]], add=True)
        return 0

    lax.fori_loop(0, NCH, edge_chunk, 0)
    plsc.subcore_barrier()

    pltpu.sync_copy(
        acc.at[pl.ds(s * ROWS_PER_TILE, ROWS_PER_TILE)],
        out_hbm.at[c, pl.ds(s * ROWS_PER_TILE, ROWS_PER_TILE)],
    )


@functools.partial(
    pl.kernel,
    out_type=jax.ShapeDtypeStruct((NC, N_PAD, D), jnp.float32),
    mesh=_mesh,
    scratch_types=[
        pltpu.VMEM((NCH, CH), jnp.int32),
        pltpu.VMEM((NCH, CH), jnp.int32),
        pltpu.VMEM((CH, D), jnp.float32),
        pltpu.VMEM_SHARED((N_PAD, D), jnp.float32),
        pltpu.SemaphoreType.DMA,
    ],
)
def _scatter_kernel(y_hbm, src_hbm, dst_hbm, out_hbm, src_v, dst_v,
                    rows_v, acc, sem):
    c = lax.axis_index("c")
    s = lax.axis_index("s")
    w = c * NS + s

    zv = jnp.zeros((16,), jnp.float32)

    def zbody(i, _):
        for k in range(D // 16):
            rows_v[i, pl.ds(k * 16, 16)] = zv
        return 0

    lax.fori_loop(0, CH, zbody, 0)
    for k in range(ROWS_PER_TILE // CH):
        pltpu.sync_copy(rows_v,
                        acc.at[pl.ds(s * ROWS_PER_TILE + k * CH, CH)])
    plsc.subcore_barrier()

    pltpu.sync_copy(src_hbm.at[w], src_v)
    pltpu.sync_copy(dst_hbm.at[w], dst_v)

    def edge_chunk(j, _):
        pltpu.async_copy(y_hbm.at[src_v.at[j]], rows_v, sem).wait()
        pltpu.sync_copy(rows_v, acc.at[dst_v.at[j]], add=True)
        return 0

    lax.fori_loop(0, NCH, edge_chunk, 0)
    plsc.subcore_barrier()

    pltpu.sync_copy(
        acc.at[pl.ds(s * ROWS_PER_TILE, ROWS_PER_TILE)],
        out_hbm.at[c, pl.ds(s * ROWS_PER_TILE, ROWS_PER_TILE)],
    )


_GRID = 8
_BM = N_PAD // _GRID


def _prep_body(degp_ref, x_ref, w0_ref, dis_ref, y0_ref):
    deg = degp_ref[0, :, 0:1] + degp_ref[1, :, 0:1] + 1.0
    dis = 1.0 / jnp.sqrt(deg)
    dis_ref[...] = dis
    y0_ref[...] = dis * jnp.dot(x_ref[...], w0_ref[...],
                                preferred_element_type=jnp.float32)


def _mid_body(sp_ref, y0_ref, dis_ref, p_ref, w1_ref, y1_ref):
    dis = dis_ref[...]
    agg = dis * (sp_ref[0] + sp_ref[1] + y0_ref[...]) + p_ref[0:1, :]
    x1 = jnp.where(agg >= 0, agg, 0.05 * agg)
    inv_std = p_ref[1:2, :] / jnp.sqrt(p_ref[4:5, :] + EPS_BN)
    xbn = (x1 - p_ref[3:4, :]) * inv_std + p_ref[2:3, :]
    y1_ref[...] = dis * jnp.dot(xbn, w1_ref[...],
                                preferred_element_type=jnp.float32)


def _out_body(sp_ref, y1_ref, dis_ref, p_ref, o_ref):
    o_ref[...] = (dis_ref[...] * (sp_ref[0] + sp_ref[1] + y1_ref[...])
                  + p_ref[5:6, :])


_prep_call = pl.pallas_call(
    _prep_body,
    grid=(_GRID,),
    in_specs=[
        pl.BlockSpec((NC, _BM, 16), lambda i: (0, i, 0)),
        pl.BlockSpec((_BM, D), lambda i: (i, 0)),
        pl.BlockSpec((D, D), lambda i: (0, 0)),
    ],
    out_specs=[
        pl.BlockSpec((_BM, 1), lambda i: (i, 0)),
        pl.BlockSpec((_BM, D), lambda i: (i, 0)),
    ],
    out_shape=[
        jax.ShapeDtypeStruct((N_PAD, 1), jnp.float32),
        jax.ShapeDtypeStruct((N_PAD, D), jnp.float32),
    ],
)

_mid_call = pl.pallas_call(
    _mid_body,
    grid=(_GRID,),
    in_specs=[
        pl.BlockSpec((NC, _BM, D), lambda i: (0, i, 0)),
        pl.BlockSpec((_BM, D), lambda i: (i, 0)),
        pl.BlockSpec((_BM, 1), lambda i: (i, 0)),
        pl.BlockSpec((8, D), lambda i: (0, 0)),
        pl.BlockSpec((D, D), lambda i: (0, 0)),
    ],
    out_specs=pl.BlockSpec((_BM, D), lambda i: (i, 0)),
    out_shape=jax.ShapeDtypeStruct((N_PAD, D), jnp.float32),
)

_out_call = pl.pallas_call(
    _out_body,
    grid=(_GRID,),
    in_specs=[
        pl.BlockSpec((NC, _BM, D), lambda i: (0, i, 0)),
        pl.BlockSpec((_BM, D), lambda i: (i, 0)),
        pl.BlockSpec((_BM, 1), lambda i: (i, 0)),
        pl.BlockSpec((8, D), lambda i: (0, 0)),
    ],
    out_specs=pl.BlockSpec((_BM, D), lambda i: (i, 0)),
    out_shape=jax.ShapeDtypeStruct((N_PAD, D), jnp.float32),
)


@jax.jit
def kernel(node_feat, edge_index, W0, b0, gamma, beta, running_mean,
           running_var, W1, b1):
    ei = edge_index.astype(jnp.int32)
    pad = E_PAD - E
    src_pad = jnp.concatenate([ei[0], jnp.zeros((pad,), jnp.int32)])
    dst_pad = jnp.concatenate([ei[1], jnp.full((pad,), N, jnp.int32)])
    src3 = src_pad.reshape(NW, NCH, CH)
    dst3 = dst_pad.reshape(NW, NCH, CH)

    params = jnp.stack([
        b0, gamma, beta, running_mean, running_var, b1,
        jnp.zeros_like(b0), jnp.zeros_like(b0),
    ])

    x_pad = jnp.concatenate(
        [node_feat, jnp.zeros((N_PAD - N, D), node_feat.dtype)])

    degp = _deg_kernel(dst3)
    dis, y0 = _prep_call(degp, x_pad, W0)
    s0 = _scatter_kernel(y0, src3, dst3)
    y1 = _mid_call(s0, y0, dis, params, W1)
    s1 = _scatter_kernel(y1, src3, dst3)
    return _out_call(s1, y1, dis, params)[:N]


# speedup vs baseline: 2.4933x; 2.4933x over previous
import functools

import jax
import jax.numpy as jnp
from jax import lax
from jax.experimental import pallas as pl
from jax.experimental.pallas import tpu as pltpu
from jax.experimental.pallas import tpu_sc as plsc

N = 10000
E = 320000
D = 128
EPS_BN = 1e-5

NC = 2
NS = 16
NW = NC * NS

CH = 128
NCH = 80
TOT_CH = NCH * NW
E_PAD = TOT_CH * CH
NCH0 = 80
NCH1 = 80
NCH_MAX = max(NCH0, NCH1)
N_PAD = 10240
ROWS_PER_TILE = N_PAD // NS

_mesh = plsc.VectorSubcoreMesh(core_axis_name="c", subcore_axis_name="s")


def _zero_rows(zbuf, lanes_per_row):
    zv = jnp.zeros((16,), jnp.float32)

    def body(i, _):
        for k in range(lanes_per_row // 16):
            zbuf[i, pl.ds(k * 16, 16)] = zv
        return 0

    lax.fori_loop(0, CH, body, 0)


@functools.partial(
    pl.kernel,
    out_type=jax.ShapeDtypeStruct((NC, N_PAD, 16), jnp.float32),
    mesh=_mesh,
    scratch_types=[
        pltpu.VMEM((NCH, CH), jnp.int32),
        pltpu.VMEM((CH, 16), jnp.float32),
        pltpu.VMEM((CH, 16), jnp.float32),
        pltpu.VMEM_SHARED((N_PAD, 16), jnp.float32),
        pltpu.SemaphoreType.DMA,
    ],
)
def _deg_kernel(dst_hbm, out_hbm, dst_v, ones_v, zbuf, acc, sem):
    c = lax.axis_index("c")
    s = lax.axis_index("s")
    w = c * NS + s

    ov = jnp.ones((16,), jnp.float32)

    def fill(i, _):
        ones_v[i, :] = ov
        return 0

    lax.fori_loop(0, CH, fill, 0)
    _zero_rows(zbuf, 16)
    for k in range(ROWS_PER_TILE // CH):
        pltpu.sync_copy(zbuf, acc.at[pl.ds(s * ROWS_PER_TILE + k * CH, CH)])
    plsc.subcore_barrier()

    pltpu.sync_copy(dst_hbm.at[w], dst_v)

    def edge_chunk(j, _):
        pltpu.sync_copy(ones_v, acc.at[dst_v.at[j]], add=True)
        return 0

    lax.fori_loop(0, NCH, edge_chunk, 0)
    plsc.subcore_barrier()

    pltpu.sync_copy(
        acc.at[pl.ds(s * ROWS_PER_TILE, ROWS_PER_TILE)],
        out_hbm.at[c, pl.ds(s * ROWS_PER_TILE, ROWS_PER_TILE)],
    )


@functools.partial(
    pl.kernel,
    out_type=jax.ShapeDtypeStruct((NC, N_PAD, D), jnp.float32),
    mesh=_mesh,
    scratch_types=[
        pltpu.VMEM((NCH, CH), jnp.int32),
        pltpu.VMEM((NCH, CH), jnp.int32),
        pltpu.VMEM((CH, D), jnp.float32),
        pltpu.VMEM_SHARED((N_PAD, D), jnp.float32),
        pltpu.SemaphoreType.DMA,
    ],
)
def _scatter_kernel(y_hbm, src_hbm, dst_hbm, out_hbm, src_v, dst_v,
                    rows_v, acc, sem):
    c = lax.axis_index("c")
    s = lax.axis_index("s")
    w = c * NS + s

    zv = jnp.zeros((16,), jnp.float32)

    def zbody(i, _):
        for k in range(D // 16):
            rows_v[i, pl.ds(k * 16, 16)] = zv
        return 0

    lax.fori_loop(0, CH, zbody, 0)
    for k in range(ROWS_PER_TILE // CH):
        pltpu.sync_copy(rows_v,
                        acc.at[pl.ds(s * ROWS_PER_TILE + k * CH, CH)])
    plsc.subcore_barrier()

    pltpu.sync_copy(src_hbm.at[w], src_v)
    pltpu.sync_copy(dst_hbm.at[w], dst_v)

    def edge_chunk(j, _):
        pltpu.async_copy(y_hbm.at[src_v.at[j]], rows_v, sem).wait()
        pltpu.sync_copy(rows_v, acc.at[dst_v.at[j]], add=True)
        return 0

    lax.fori_loop(0, NCH, edge_chunk, 0)
    plsc.subcore_barrier()

    pltpu.sync_copy(
        acc.at[pl.ds(s * ROWS_PER_TILE, ROWS_PER_TILE)],
        out_hbm.at[c, pl.ds(s * ROWS_PER_TILE, ROWS_PER_TILE)],
    )


_GRID = 8
_BM = N_PAD // _GRID


def _prep_body(degp_ref, x_ref, w0_ref, dis_ref, y0_ref):
    deg = degp_ref[0, :, 0:1] + degp_ref[1, :, 0:1] + 1.0
    dis = 1.0 / jnp.sqrt(deg)
    dis_ref[...] = dis
    y0_ref[...] = dis * jnp.dot(x_ref[...], w0_ref[...],
                                preferred_element_type=jnp.float32)


def _mid_body(sp_ref, y0_ref, dis_ref, p_ref, w1_ref, y1_ref):
    dis = dis_ref[...]
    agg = dis * (sp_ref[0] + sp_ref[1] + y0_ref[...]) + p_ref[0:1, :]
    x1 = jnp.where(agg >= 0, agg, 0.05 * agg)
    inv_std = p_ref[1:2, :] / jnp.sqrt(p_ref[4:5, :] + EPS_BN)
    xbn = (x1 - p_ref[3:4, :]) * inv_std + p_ref[2:3, :]
    y1_ref[...] = dis * jnp.dot(xbn, w1_ref[...],
                                preferred_element_type=jnp.float32)


def _out_body(sp_ref, y1_ref, dis_ref, p_ref, o_ref):
    o_ref[...] = (dis_ref[...] * (sp_ref[0] + sp_ref[1] + y1_ref[...])
                  + p_ref[5:6, :])


_prep_call = pl.pallas_call(
    _prep_body,
    grid=(_GRID,),
    in_specs=[
        pl.BlockSpec((NC, _BM, 16), lambda i: (0, i, 0)),
        pl.BlockSpec((_BM, D), lambda i: (i, 0)),
        pl.BlockSpec((D, D), lambda i: (0, 0)),
    ],
    out_specs=[
        pl.BlockSpec((_BM, 1), lambda i: (i, 0)),
        pl.BlockSpec((_BM, D), lambda i: (i, 0)),
    ],
    out_shape=[
        jax.ShapeDtypeStruct((N_PAD, 1), jnp.float32),
        jax.ShapeDtypeStruct((N_PAD, D), jnp.float32),
    ],
)

_mid_call = pl.pallas_call(
    _mid_body,
    grid=(_GRID,),
    in_specs=[
        pl.BlockSpec((NC, _BM, D), lambda i: (0, i, 0)),
        pl.BlockSpec((_BM, D), lambda i: (i, 0)),
        pl.BlockSpec((_BM, 1), lambda i: (i, 0)),
        pl.BlockSpec((8, D), lambda i: (0, 0)),
        pl.BlockSpec((D, D), lambda i: (0, 0)),
    ],
    out_specs=pl.BlockSpec((_BM, D), lambda i: (i, 0)),
    out_shape=jax.ShapeDtypeStruct((N_PAD, D), jnp.float32),
)

_out_call = pl.pallas_call(
    _out_body,
    grid=(_GRID,),
    in_specs=[
        pl.BlockSpec((NC, _BM, D), lambda i: (0, i, 0)),
        pl.BlockSpec((_BM, D), lambda i: (i, 0)),
        pl.BlockSpec((_BM, 1), lambda i: (i, 0)),
        pl.BlockSpec((8, D), lambda i: (0, 0)),
    ],
    out_specs=pl.BlockSpec((_BM, D), lambda i: (i, 0)),
    out_shape=jax.ShapeDtypeStruct((N_PAD, D), jnp.float32),
)


@jax.jit
def kernel(node_feat, edge_index, W0, b0, gamma, beta, running_mean,
           running_var, W1, b1):
    ei = edge_index.astype(jnp.int32)
    pad = E_PAD - E
    dump = N + (jnp.arange(pad, dtype=jnp.int32) % (N_PAD - N))
    src_pad = jnp.concatenate([ei[0], dump])
    dst_pad = jnp.concatenate([ei[1], dump])
    src3 = src_pad.reshape(NW, NCH, CH)
    dst3 = dst_pad.reshape(NW, NCH, CH)

    params = jnp.stack([
        b0, gamma, beta, running_mean, running_var, b1,
        jnp.zeros_like(b0), jnp.zeros_like(b0),
    ])

    x_pad = jnp.concatenate(
        [node_feat, jnp.zeros((N_PAD - N, D), node_feat.dtype)])

    degp = _deg_kernel(dst3)
    dis, y0 = _prep_call(degp, x_pad, W0)
    s0 = _scatter_kernel(y0, src3, dst3)
    y1 = _mid_call(s0, y0, dis, params, W1)
    s1 = _scatter_kernel(y1, src3, dst3)
    return _out_call(s1, y1, dis, params)[:N]
